# baseline (speedup 1.0000x reference)
import jax
import jax.numpy as jnp
from jax import lax
from jax.experimental import pallas as pl
from jax.experimental.pallas import tpu as pltpu

COMPRESSION_RATIO = 0.8
MAX_TOKENS = 2048


def _topk_kernel(scores_ref, idx_ref):
    B, N = scores_ref.shape
    K = idx_ref.shape[1]
    s0 = scores_ref[...]
    pos = lax.broadcasted_iota(jnp.int32, (1, N), 1)
    idx0 = jnp.broadcast_to(pos, (B, N)).astype(jnp.int32)
    bits = pltpu.bitcast(s0, jnp.int32)
    keys = jnp.where(bits < 0, jnp.int32(-0x80000000) - bits, bits)
    state0 = jnp.concatenate([keys, idx0], axis=0)

    def stage(carry):
        k, j, st = carry
        bit_j = (pos & j) != 0
        desc = (pos & k) == 0
        back = pltpu.roll(st, j, 1)
        fwd = pltpu.roll(st, N - j, 1)
        partner = jnp.where(bit_j, back, fwd)
        ks, idx = st[:B], st[B:]
        kp, ip = partner[:B], partner[B:]
        self_better = (ks > kp) | ((ks == kp) & (idx < ip))
        keep_self = self_better == (bit_j == (~desc))
        keep2 = jnp.concatenate([keep_self, keep_self], axis=0)
        st = jnp.where(keep2, st, partner)
        j = j // 2
        k = jnp.where(j == 0, k * 2, k)
        j = jnp.where(j == 0, k // 2, j)
        return (k, j, st)

    def cond(carry):
        return carry[0] <= N

    init = (jnp.int32(2), jnp.int32(1), state0)
    _, _, st = lax.while_loop(cond, stage, init)
    idx_ref[...] = st[B:, :K]


def _topk_indices(scores, max_k):
    B, N = scores.shape
    return pl.pallas_call(
        _topk_kernel,
        out_shape=jax.ShapeDtypeStruct((B, max_k), jnp.int32),
    )(scores)


def kernel(padded_bag, key_padding_mask, text_feature_batch, W_q, b_q, W_k, b_k):
    B, N, D = padded_bag.shape
    num_patches = (~key_padding_mask).sum(axis=1)
    k_per_bag = (num_patches.astype(jnp.float32) * COMPRESSION_RATIO).astype(jnp.int32)
    k_per_bag = jnp.clip(k_per_bag, 1, MAX_TOKENS)
    k_per_bag = jnp.minimum(k_per_bag, num_patches.astype(jnp.int32))
    k_per_bag = jnp.where(k_per_bag == 0, 1, k_per_bag)
    max_k = min(max(1, min(int(N * COMPRESSION_RATIO), MAX_TOKENS)), N)

    text_q = text_feature_batch @ W_q.T + b_q
    patches_k = padded_bag @ W_k.T + b_k
    scores = jnp.einsum('bd,bnd->bn', text_q, patches_k)
    scores = jnp.where(key_padding_mask, -jnp.inf, scores)

    idx = _topk_indices(scores, max_k)
    compressed = jnp.take_along_axis(padded_bag, idx[:, :, None], axis=1)
    new_mask = jnp.arange(max_k)[None, :] >= k_per_bag[:, None]
    return (compressed, new_mask)


# speedup vs baseline: 1.0307x; 1.0307x over previous
import jax
import jax.numpy as jnp
from jax import lax
from jax.experimental import pallas as pl
from jax.experimental.pallas import tpu as pltpu

COMPRESSION_RATIO = 0.8
MAX_TOKENS = 2048


def _topk_kernel(scores_ref, idx_ref):
    B, N = scores_ref.shape
    K = idx_ref.shape[1]
    s0 = scores_ref[...]
    pos = lax.broadcasted_iota(jnp.int32, (1, N), 1)
    idx0 = jnp.broadcast_to(pos, (B, N)).astype(jnp.int32)

    def stage(carry):
        k, j, s, idx = carry
        bit_j = (pos & j) != 0
        desc = (pos & k) == 0
        back = pltpu.roll(s, j, 1)
        fwd = pltpu.roll(s, N - j, 1)
        sp = jnp.where(bit_j, back, fwd)
        backi = pltpu.roll(idx, j, 1)
        fwdi = pltpu.roll(idx, N - j, 1)
        ip = jnp.where(bit_j, backi, fwdi)
        self_better = (s > sp) | ((s == sp) & (idx < ip))
        keep_self = self_better == (bit_j == (~desc))
        s = jnp.where(keep_self, s, sp)
        idx = jnp.where(keep_self, idx, ip)
        j = j // 2
        k = jnp.where(j == 0, k * 2, k)
        j = jnp.where(j == 0, k // 2, j)
        return (k, j, s, idx)

    def cond(carry):
        return carry[0] <= N

    init = (jnp.int32(2), jnp.int32(1), s0, idx0)
    _, _, _, idx = lax.while_loop(cond, stage, init)
    idx_ref[...] = idx[:, :K]


def _topk_indices(scores, max_k):
    B, N = scores.shape
    return pl.pallas_call(
        _topk_kernel,
        out_shape=jax.ShapeDtypeStruct((B, max_k), jnp.int32),
    )(scores)


def kernel(padded_bag, key_padding_mask, text_feature_batch, W_q, b_q, W_k, b_k):
    B, N, D = padded_bag.shape
    num_patches = (~key_padding_mask).sum(axis=1)
    k_per_bag = (num_patches.astype(jnp.float32) * COMPRESSION_RATIO).astype(jnp.int32)
    k_per_bag = jnp.clip(k_per_bag, 1, MAX_TOKENS)
    k_per_bag = jnp.minimum(k_per_bag, num_patches.astype(jnp.int32))
    k_per_bag = jnp.where(k_per_bag == 0, 1, k_per_bag)
    max_k = min(max(1, min(int(N * COMPRESSION_RATIO), MAX_TOKENS)), N)

    text_q = text_feature_batch @ W_q.T + b_q
    patches_k = padded_bag @ W_k.T + b_k
    scores = jnp.einsum('bd,bnd->bn', text_q, patches_k)
    scores = jnp.where(key_padding_mask, -jnp.inf, scores)

    idx = _topk_indices(scores, max_k)
    compressed = jnp.take_along_axis(padded_bag, idx[:, :, None], axis=1)
    new_mask = jnp.arange(max_k)[None, :] >= k_per_bag[:, None]
    return (compressed, new_mask)
